# Initial kernel scaffold; baseline (speedup 1.0000x reference)
import jax
import jax.numpy as jnp
from jax.experimental import pallas as pl


def kernel(emb, adj, batch, prot_feature, c1_w, c1_b, c2_w, c2_b, fc1_w, fc1_b, fc2_w, fc2_b):
    raise NotImplementedError("write your pallas kernel here")


# speedup vs baseline: 2.2596x; 2.2596x over previous
import jax
import jax.numpy as jnp
from jax import lax
from jax.experimental import pallas as pl
from jax.experimental.pallas import tpu as pltpu


def _gib_kernel(emb_ref, x_ref, adj_ref, bcol_ref, brow_ref,
                c1w_ref, c1b_ref, dw_ref, db_ref,
                f1w_ref, f1b_ref, f2w_ref, f2b_ref,
                out_ref, pos_ref, gra_ref, assign_ref, pen_ref,
                *, nc, s, inv_nodes, inv_groups):
    r = nc * s
    p = x_ref.shape[-1]

    e = emb_ref[...]
    h = jnp.dot(e.astype(jnp.bfloat16), c1w_ref[...],
                preferred_element_type=jnp.float32) + c1b_ref[...]
    h = jnp.maximum(h, 0.0)
    diff = jnp.dot(h, dw_ref[...],
                   preferred_element_type=jnp.float32) + db_ref[...]

    a0 = 1.0 / (1.0 + jnp.exp(diff))
    col = lax.broadcasted_iota(jnp.int32, (r, 2), 1)
    assign_ref[...] = jnp.where(col == 0, a0, 1.0 - a0).reshape(1, r, 2)

    x = x_ref[...]
    pos = jnp.sum((a0 * x).reshape(nc, s, p), axis=1) * inv_nodes
    gra = jnp.sum(x.reshape(nc, s, p), axis=1) * inv_nodes
    pos_ref[...] = pos.reshape(1, nc, p)
    gra_ref[...] = gra.reshape(1, nc, p)

    adj = adj_ref[...]
    b0 = a0.reshape(nc, s)
    t0 = jnp.dot(b0, adj, preferred_element_type=jnp.float32)
    t1 = jnp.sum(adj, axis=0, keepdims=True) - t0
    e00 = jnp.sum(t0 * b0, axis=1, keepdims=True)
    e01 = jnp.sum(t0, axis=1, keepdims=True) - e00
    e10 = jnp.sum(t1 * b0, axis=1, keepdims=True)
    e11 = jnp.sum(t1, axis=1, keepdims=True) - e10
    n0 = jnp.maximum(jnp.abs(e00) + jnp.abs(e01), 1e-5)
    n1 = jnp.maximum(jnp.abs(e10) + jnp.abs(e11), 1e-5)
    d0 = e00 / n0
    d1 = e11 / n1
    mse = 0.5 * ((d0 - 1.0) ** 2 + (d1 - 1.0) ** 2)

    bc = bcol_ref[0]
    eq = (bc == brow_ref[...]).astype(jnp.float32)
    counts = jnp.sum(eq, axis=1, keepdims=True)
    pen_ref[...] = jnp.sum((inv_groups / counts) * mse).reshape(1, 1, 1)

    hh = jnp.dot(pos, f1w_ref[...],
                 preferred_element_type=jnp.float32) + f1b_ref[...]
    hh = jnp.maximum(hh, 0.0)
    out = jnp.dot(hh, f2w_ref[...],
                  preferred_element_type=jnp.float32) + f2b_ref[...]
    out_ref[...] = out.reshape(1, nc, -1)


def kernel(emb, adj, batch, prot_feature,
           c1_w, c1_b, c2_w, c2_b, fc1_w, fc1_b, fc2_w, fc2_b):
    N, S, DIN = emb.shape
    P = prot_feature.shape[-1]
    H = fc2_w.shape[-1]
    NS = N * S
    G = 4
    NC = N // G
    R = NS // G
    NUM_GROUPS = 4

    emb2d = emb.reshape(NS, DIN)
    x2d = prot_feature.reshape(NS, P)
    c1w_bf = c1_w.astype(jnp.bfloat16)
    dw = c2_w[:, 1:2] - c2_w[:, 0:1]
    db = c2_b[:, 1:2] - c2_b[:, 0:1]
    bcol3 = batch.astype(jnp.int32).reshape(G, NC, 1)
    brow = batch.astype(jnp.int32).reshape(1, N)

    import functools
    body = functools.partial(_gib_kernel, nc=NC, s=S,
                             inv_nodes=1.0 / S, inv_groups=1.0 / NUM_GROUPS)

    out_shapes = (
        jax.ShapeDtypeStruct((G, NC, H), jnp.float32),
        jax.ShapeDtypeStruct((G, NC, P), jnp.float32),
        jax.ShapeDtypeStruct((G, NC, P), jnp.float32),
        jax.ShapeDtypeStruct((G, R, 2), jnp.float32),
        jax.ShapeDtypeStruct((G, 1, 1), jnp.float32),
    )

    out3, pos3, gra3, assign3, pen3 = pl.pallas_call(
        body,
        out_shape=out_shapes,
        grid=(G,),
        in_specs=[
            pl.BlockSpec((R, DIN), lambda i: (i, 0)),
            pl.BlockSpec((R, P), lambda i: (i, 0)),
            pl.BlockSpec((S, S), lambda i: (0, 0)),
            pl.BlockSpec((1, NC, 1), lambda i: (i, 0, 0)),
            pl.BlockSpec((1, N), lambda i: (0, 0)),
            pl.BlockSpec((DIN, P), lambda i: (0, 0)),
            pl.BlockSpec((1, P), lambda i: (0, 0)),
            pl.BlockSpec((P, 1), lambda i: (0, 0)),
            pl.BlockSpec((1, 1), lambda i: (0, 0)),
            pl.BlockSpec((P, P), lambda i: (0, 0)),
            pl.BlockSpec((1, P), lambda i: (0, 0)),
            pl.BlockSpec((P, H), lambda i: (0, 0)),
            pl.BlockSpec((1, H), lambda i: (0, 0)),
        ],
        out_specs=[
            pl.BlockSpec((1, NC, H), lambda i: (i, 0, 0)),
            pl.BlockSpec((1, NC, P), lambda i: (i, 0, 0)),
            pl.BlockSpec((1, NC, P), lambda i: (i, 0, 0)),
            pl.BlockSpec((1, R, 2), lambda i: (i, 0, 0)),
            pl.BlockSpec((1, 1, 1), lambda i: (i, 0, 0)),
        ],
        compiler_params=pltpu.CompilerParams(
            dimension_semantics=("parallel",)),
    )(emb2d, x2d, adj, bcol3, brow,
      c1w_bf, c1_b, dw, db, fc1_w, fc1_b, fc2_w, fc2_b)

    return (out3.reshape(N, H), pos3.reshape(N, P), gra3.reshape(N, P),
            jnp.sum(pen3), assign3.reshape(N, S, 2))


# speedup vs baseline: 3.9618x; 1.7533x over previous
import functools

import jax
import jax.numpy as jnp
from jax import lax
from jax.experimental import pallas as pl
from jax.experimental.pallas import tpu as pltpu


def _gib_kernel(emb_ref, x_ref, adj_ref, brow_ref,
                c1w_ref, c1b_ref, c2w_ref, c2b_ref,
                f1w_ref, f1b_ref, f2w_ref, f2b_ref,
                out_ref, pos_ref, gra_ref, assign_ref, pen_ref,
                c1wbf_ref,
                *, nc, s, n, inv_nodes, inv_groups):
    i = pl.program_id(0)
    r = nc * s
    p = x_ref.shape[-1]

    @pl.when(i == 0)
    def _():
        c1wbf_ref[...] = c1w_ref[...].astype(jnp.bfloat16)

    e = emb_ref[...]
    h = jnp.dot(e.astype(jnp.bfloat16), c1wbf_ref[...],
                preferred_element_type=jnp.float32) + c1b_ref[...]
    h = jnp.maximum(h, 0.0)
    dw = c2w_ref[:, 1:2] - c2w_ref[:, 0:1]
    db = c2b_ref[:, 1:2] - c2b_ref[:, 0:1]
    diff = jnp.dot(h, dw, preferred_element_type=jnp.float32) + db

    a0 = 1.0 / (1.0 + jnp.exp(diff))
    col = lax.broadcasted_iota(jnp.int32, (r, 2), 1)
    assign = jnp.where(col == 0, a0, 1.0 - a0)
    assign_ref[...] = assign.reshape(nc, s, 2)

    x = x_ref[...]
    pos = jnp.sum((a0 * x).reshape(nc, s, p), axis=1) * inv_nodes
    gra = jnp.sum(x.reshape(nc, s, p), axis=1) * inv_nodes
    pos_ref[...] = pos
    gra_ref[...] = gra

    adj = adj_ref[...]
    b0 = a0.reshape(nc, s)
    t0 = jnp.dot(b0, adj, preferred_element_type=jnp.float32)
    t1 = jnp.sum(adj, axis=0, keepdims=True) - t0
    e00 = jnp.sum(t0 * b0, axis=1, keepdims=True)
    e01 = jnp.sum(t0, axis=1, keepdims=True) - e00
    e10 = jnp.sum(t1 * b0, axis=1, keepdims=True)
    e11 = jnp.sum(t1, axis=1, keepdims=True) - e10
    n0 = jnp.maximum(jnp.abs(e00) + jnp.abs(e01), 1e-5)
    n1 = jnp.maximum(jnp.abs(e10) + jnp.abs(e11), 1e-5)
    d0 = e00 / n0
    d1 = e11 / n1
    mse = 0.5 * ((d0 - 1.0) ** 2 + (d1 - 1.0) ** 2)

    brow = brow_ref[...]
    bcol = brow.reshape(n, 1)
    eq = (bcol == brow).astype(jnp.float32)
    counts = jnp.sum(eq, axis=1, keepdims=True)
    rowi = lax.broadcasted_iota(jnp.int32, (nc, n), 0)
    coli = lax.broadcasted_iota(jnp.int32, (nc, n), 1)
    sel = (coli == rowi + i * nc).astype(jnp.float32)
    wchunk = jnp.dot(sel, inv_groups / counts,
                     preferred_element_type=jnp.float32)
    part = jnp.sum(wchunk * mse).reshape(1, 1)

    @pl.when(i == 0)
    def _():
        pen_ref[...] = part

    @pl.when(i > 0)
    def _():
        pen_ref[...] += part

    hh = jnp.dot(pos, f1w_ref[...],
                 preferred_element_type=jnp.float32) + f1b_ref[...]
    hh = jnp.maximum(hh, 0.0)
    out_ref[...] = jnp.dot(hh, f2w_ref[...],
                           preferred_element_type=jnp.float32) + f2b_ref[...]


def kernel(emb, adj, batch, prot_feature,
           c1_w, c1_b, c2_w, c2_b, fc1_w, fc1_b, fc2_w, fc2_b):
    N, S, DIN = emb.shape
    P = prot_feature.shape[-1]
    H = fc2_w.shape[-1]
    NS = N * S
    G = 2
    NC = N // G
    R = NS // G
    NUM_GROUPS = 4

    emb2d = emb.reshape(NS, DIN)
    x2d = prot_feature.reshape(NS, P)
    brow = batch.astype(jnp.int32).reshape(1, N)

    body = functools.partial(_gib_kernel, nc=NC, s=S, n=N,
                             inv_nodes=1.0 / S, inv_groups=1.0 / NUM_GROUPS)

    out_shapes = (
        jax.ShapeDtypeStruct((N, H), jnp.float32),
        jax.ShapeDtypeStruct((N, P), jnp.float32),
        jax.ShapeDtypeStruct((N, P), jnp.float32),
        jax.ShapeDtypeStruct((N, S, 2), jnp.float32),
        jax.ShapeDtypeStruct((1, 1), jnp.float32),
    )

    out, pos, gra, assign, pen = pl.pallas_call(
        body,
        out_shape=out_shapes,
        grid=(G,),
        in_specs=[
            pl.BlockSpec((R, DIN), lambda i: (i, 0)),
            pl.BlockSpec((R, P), lambda i: (i, 0)),
            pl.BlockSpec((S, S), lambda i: (0, 0)),
            pl.BlockSpec((1, N), lambda i: (0, 0)),
            pl.BlockSpec((DIN, P), lambda i: (0, 0)),
            pl.BlockSpec((1, P), lambda i: (0, 0)),
            pl.BlockSpec((P, 2), lambda i: (0, 0)),
            pl.BlockSpec((1, 2), lambda i: (0, 0)),
            pl.BlockSpec((P, P), lambda i: (0, 0)),
            pl.BlockSpec((1, P), lambda i: (0, 0)),
            pl.BlockSpec((P, H), lambda i: (0, 0)),
            pl.BlockSpec((1, H), lambda i: (0, 0)),
        ],
        out_specs=[
            pl.BlockSpec((NC, H), lambda i: (i, 0)),
            pl.BlockSpec((NC, P), lambda i: (i, 0)),
            pl.BlockSpec((NC, P), lambda i: (i, 0)),
            pl.BlockSpec((NC, S, 2), lambda i: (i, 0, 0)),
            pl.BlockSpec((1, 1), lambda i: (0, 0)),
        ],
        scratch_shapes=[pltpu.VMEM((DIN, P), jnp.bfloat16)],
        compiler_params=pltpu.CompilerParams(
            dimension_semantics=("arbitrary",)),
    )(emb2d, x2d, adj, brow,
      c1_w, c1_b, c2_w, c2_b, fc1_w, fc1_b, fc2_w, fc2_b)

    return out, pos, gra, pen[0, 0], assign
